# baseline (speedup 1.0000x reference)
import jax
import jax.numpy as jnp
from jax import lax
from jax.experimental import pallas as pl
from jax.experimental.pallas import tpu as pltpu
from jax.experimental.pallas import tpu_sc as plsc

_K = 256
_M, _N = 128, 32768

_M_SC = 64
_M_TC = _M - _M_SC
_TC_BLOCK = 32

_NC, _NS = 2, 16
_NW = _NC * _NS
_RPW = _M_SC // _NW
_NV = _N // 16

_ABS = 0x7FFFFFFF


def _sc_body(x_hbm, o_hbm, x_v, cand_v, cidx_v, hist, hist16, sem):
    del sem
    cidx = lax.axis_index("c")
    sidx = lax.axis_index("s")
    wid = sidx * _NC + cidx
    iota = lax.iota(jnp.int32, 16)
    iota256 = iota * 256
    iota16 = iota * 16
    ones = jnp.ones((16,), jnp.int32)
    zeros = jnp.zeros((16,), jnp.int32)
    padidx = jnp.int32(_N) + iota

    def choose256():
        best_d = jnp.int32(-1)
        s_best = jnp.int32(0)
        h_best = jnp.int32(0)
        carry = jnp.int32(0)
        for c in range(15, -1, -1):
            acc = jnp.zeros((16,), jnp.int32)
            for l in range(16):
                acc = acc + hist[pl.ds(l * 256 + c * 16, 16)]
            rev = lax.rev(acc, (0,))
            cs = plsc.cumsum(rev) + carry
            dd = jnp.int32(c * 16 + 15) - iota
            sel = cs >= _K
            d_c = jnp.max(jnp.where(sel, dd, jnp.int32(-1)))
            s_c = jnp.sum(jnp.where(dd == d_c, cs, jnp.int32(0)))
            h_c = jnp.sum(jnp.where(dd == d_c, rev, jnp.int32(0)))
            take = (best_d < 0) & (d_c >= 0)
            best_d = jnp.where(take, d_c, best_d)
            s_best = jnp.where(take, s_c, s_best)
            h_best = jnp.where(take, h_c, h_best)
            carry = carry + jnp.sum(acc)
        return best_d, jnp.int32(_K) - (s_best - h_best)

    def choose16(k_rem):
        acc = jnp.zeros((16,), jnp.int32)
        for l in range(16):
            acc = acc + hist16[pl.ds(l * 16, 16)]
        rev = lax.rev(acc, (0,))
        csum = plsc.cumsum(rev)
        dd = jnp.int32(15) - iota
        sel = csum >= k_rem
        d = jnp.max(jnp.where(sel, dd, jnp.int32(-1)))
        cnt_ge = jnp.sum(jnp.where(dd == d, csum, jnp.int32(0)))
        h_d = jnp.sum(jnp.where(iota == d, acc, jnp.int32(0)))
        return d, k_rem - (cnt_ge - h_d)

    for j in range(_RPW):
        row = wid * _RPW + j
        pltpu.sync_copy(x_hbm.at[row], x_v.at[pl.ds(0, _N)])

        for l in range(256):
            hist[pl.ds(l * 16, 16)] = zeros

        def ha(i, carry):
            v = x_v[pl.ds(i * 16, 16)]
            d = (v >> 23) & 255
            plsc.addupdate_scatter(hist, [iota256 + d], ones)
            return carry

        lax.fori_loop(0, _NV, ha, jnp.int32(0))
        b0, k_rem = choose256()

        def hb(i, w):
            v = x_v[pl.ds(i * 16, 16)]
            d = (v >> 23) & 255
            x_v[pl.ds(i * 16, 16)] = jnp.where(d > b0, v, jnp.int32(0))
            mi = (d == b0).astype(jnp.int32)
            incl = plsc.cumsum(mi)
            pos = w + incl - mi
            mb = mi == 1
            plsc.store_scatter(cand_v, [pos], v, mask=mb)
            plsc.store_scatter(cidx_v, [pos], i * 16 + iota, mask=mb)
            return w + incl[15]

        n_cur = lax.fori_loop(0, _NV, hb, jnp.int32(0))
        plsc.store_scatter(cand_v, [n_cur + iota], zeros, mask=None)
        plsc.store_scatter(cidx_v, [n_cur + iota], padidx, mask=None)

        for rnd in range(6):
            sh = 19 - 4 * rnd if rnd < 5 else 0
            msk = jnp.int32(15) if rnd < 5 else jnp.int32(7)
            final = rnd == 5
            nv_cur = (n_cur + 15) >> 4

            for l in range(16):
                hist16[pl.ds(l * 16, 16)] = zeros

            def hr(i, carry, sh=sh, msk=msk):
                u = cand_v[pl.ds(i * 16, 16)] & jnp.int32(_ABS)
                d = (u >> sh) & msk
                plsc.addupdate_scatter(hist16, [iota16 + d], ones)
                return carry

            lax.fori_loop(0, nv_cur, hr, jnp.int32(0))
            d_r, k_rem = choose16(k_rem)

            if final:

                def fr(i, carry, sh=sh, msk=msk, d_r=d_r):
                    v = cand_v[pl.ds(i * 16, 16)]
                    idx = cidx_v[pl.ds(i * 16, 16)]
                    u = v & jnp.int32(_ABS)
                    d = (u >> sh) & msk
                    plsc.store_scatter(x_v, [idx], v, mask=d >= d_r)
                    return carry

                lax.fori_loop(0, nv_cur, fr, jnp.int32(0))
            else:

                def cr(i, w, sh=sh, msk=msk, d_r=d_r):
                    v = cand_v[pl.ds(i * 16, 16)]
                    idx = cidx_v[pl.ds(i * 16, 16)]
                    u = v & jnp.int32(_ABS)
                    d = (u >> sh) & msk
                    plsc.store_scatter(x_v, [idx], v, mask=d > d_r)
                    mi = (d == d_r).astype(jnp.int32)
                    incl = plsc.cumsum(mi)
                    pos = w + incl - mi
                    mb = mi == 1
                    plsc.store_scatter(cand_v, [pos], v, mask=mb)
                    plsc.store_scatter(cidx_v, [pos], idx, mask=mb)
                    return w + incl[15]

                n_cur = lax.fori_loop(0, nv_cur, cr, jnp.int32(0))
                plsc.store_scatter(cand_v, [n_cur + iota], zeros, mask=None)
                plsc.store_scatter(cidx_v, [n_cur + iota], padidx, mask=None)

        pltpu.sync_copy(x_v.at[pl.ds(0, _N)], o_hbm.at[row])


def _sc_call(xi):
    mesh = plsc.VectorSubcoreMesh(
        core_axis_name="c", subcore_axis_name="s", num_cores=_NC, num_subcores=_NS
    )
    return pl.kernel(
        _sc_body,
        out_type=jax.ShapeDtypeStruct((_M_SC, _N), jnp.int32),
        mesh=mesh,
        compiler_params=pltpu.CompilerParams(needs_layout_passes=False),
        scratch_types=[
            pltpu.VMEM((_N + 16,), jnp.int32),
            pltpu.VMEM((_N + 16,), jnp.int32),
            pltpu.VMEM((_N + 16,), jnp.int32),
            pltpu.VMEM((4096,), jnp.int32),
            pltpu.VMEM((256,), jnp.int32),
            pltpu.SemaphoreType.DMA,
        ],
    )(xi)


def _tc_block_body(x_ref, o_ref):
    r, n = x_ref.shape
    xb = x_ref[...]
    u = lax.bitcast_convert_type(xb, jnp.int32) & jnp.int32(_ABS)
    u3 = u.reshape(r, n // 128, 128)

    def _counts3(t3, sh):
        c1 = t3 | (jnp.int32(1) << sh)
        c2 = t3 | (jnp.int32(2) << sh)
        c3 = t3 | (jnp.int32(3) << sh)
        f = (
            (u3 >= c1).astype(jnp.int32)
            + jnp.where(u3 >= c2, jnp.int32(1 << 10), 0)
            + jnp.where(u3 >= c3, jnp.int32(1 << 20), 0)
        )
        s = jnp.sum(f, axis=1)
        cnt1 = jnp.sum(s & 1023, axis=-1).reshape(r, 1, 1)
        cnt2 = jnp.sum((s >> 10) & 1023, axis=-1).reshape(r, 1, 1)
        cnt3 = jnp.sum(s >> 20, axis=-1).reshape(r, 1, 1)
        return c1, c2, c3, cnt1, cnt2, cnt3

    t3 = jnp.zeros((r, 1, 1), jnp.int32)
    for i in range(15):
        sh = 29 - 2 * i
        c1, c2, c3, cnt1, cnt2, cnt3 = _counts3(t3, sh)
        t3 = jnp.where(
            cnt3 >= _K,
            c3,
            jnp.where(cnt2 >= _K, c2, jnp.where(cnt1 >= _K, c1, t3)),
        )
    cand = t3 | jnp.int32(1)
    cnt = jnp.sum((u3 >= cand).astype(jnp.int32), axis=(1, 2)).reshape(r, 1, 1)
    t3 = jnp.where(cnt >= _K, cand, t3)
    t = t3.reshape(r, 1)
    o_ref[...] = jnp.where(u >= t, xb, jnp.float32(0.0))


def _tc_call(x):
    m, n = x.shape
    r = _TC_BLOCK
    off = _M_SC // r
    return pl.pallas_call(
        _tc_block_body,
        grid=(m // r,),
        in_specs=[pl.BlockSpec((r, n), lambda i: (i, 0))],
        out_specs=pl.BlockSpec((r, n), lambda i: (i + off, 0)),
        out_shape=jax.ShapeDtypeStruct((_M, n), x.dtype),
    )(x)


def kernel(x):
    xi_sc = lax.bitcast_convert_type(x[:_M_SC], jnp.int32)
    o_sc = lax.bitcast_convert_type(_sc_call(xi_sc), jnp.float32)
    o_tc = _tc_call(x[_M_SC:])
    return lax.dynamic_update_slice(o_tc, o_sc, (0, 0))


# speedup vs baseline: 1.3503x; 1.3503x over previous
import jax
import jax.numpy as jnp
from jax import lax
from jax.experimental import pallas as pl
from jax.experimental.pallas import tpu as pltpu
from jax.experimental.pallas import tpu_sc as plsc

_K = 256
_M, _N = 128, 32768

_M_SC = 32
_M_TC = _M - _M_SC
_TC_BLOCK = 32

_NC, _NS = 2, 16
_NW = _NC * _NS
_RPW = _M_SC // _NW
_NV = _N // 16

_ABS = 0x7FFFFFFF


def _sc_body(x_hbm, o_hbm, x_v, cand_v, cidx_v, hist, hist16, sem):
    del sem
    cidx = lax.axis_index("c")
    sidx = lax.axis_index("s")
    wid = sidx * _NC + cidx
    iota = lax.iota(jnp.int32, 16)
    iota256 = iota * 256
    iota16 = iota * 16
    ones = jnp.ones((16,), jnp.int32)
    zeros = jnp.zeros((16,), jnp.int32)
    padidx = jnp.int32(_N) + iota

    def choose256():
        best_d = jnp.int32(-1)
        s_best = jnp.int32(0)
        h_best = jnp.int32(0)
        carry = jnp.int32(0)
        for c in range(15, -1, -1):
            acc = jnp.zeros((16,), jnp.int32)
            for l in range(16):
                acc = acc + hist[pl.ds(l * 256 + c * 16, 16)]
            rev = lax.rev(acc, (0,))
            cs = plsc.cumsum(rev) + carry
            dd = jnp.int32(c * 16 + 15) - iota
            sel = cs >= _K
            d_c = jnp.max(jnp.where(sel, dd, jnp.int32(-1)))
            s_c = jnp.sum(jnp.where(dd == d_c, cs, jnp.int32(0)))
            h_c = jnp.sum(jnp.where(dd == d_c, rev, jnp.int32(0)))
            take = (best_d < 0) & (d_c >= 0)
            best_d = jnp.where(take, d_c, best_d)
            s_best = jnp.where(take, s_c, s_best)
            h_best = jnp.where(take, h_c, h_best)
            carry = carry + jnp.sum(acc)
        return best_d, jnp.int32(_K) - (s_best - h_best)

    def choose16(k_rem):
        acc = jnp.zeros((16,), jnp.int32)
        for l in range(16):
            acc = acc + hist16[pl.ds(l * 16, 16)]
        rev = lax.rev(acc, (0,))
        csum = plsc.cumsum(rev)
        dd = jnp.int32(15) - iota
        sel = csum >= k_rem
        d = jnp.max(jnp.where(sel, dd, jnp.int32(-1)))
        cnt_ge = jnp.sum(jnp.where(dd == d, csum, jnp.int32(0)))
        h_d = jnp.sum(jnp.where(iota == d, acc, jnp.int32(0)))
        return d, k_rem - (cnt_ge - h_d)

    for j in range(_RPW):
        row = wid * _RPW + j
        pltpu.sync_copy(x_hbm.at[row], x_v.at[pl.ds(0, _N)])

        for l in range(256):
            hist[pl.ds(l * 16, 16)] = zeros

        def ha(i, carry):
            v = x_v[pl.ds(i * 16, 16)]
            d = (v >> 23) & 255
            plsc.addupdate_scatter(hist, [iota256 + d], ones)
            return carry

        lax.fori_loop(0, _NV, ha, jnp.int32(0))
        b0, k_rem = choose256()

        def hb(i, w):
            v = x_v[pl.ds(i * 16, 16)]
            d = (v >> 23) & 255
            x_v[pl.ds(i * 16, 16)] = jnp.where(d > b0, v, jnp.int32(0))
            mi = (d == b0).astype(jnp.int32)
            incl = plsc.cumsum(mi)
            pos = w + incl - mi
            mb = mi == 1
            plsc.store_scatter(cand_v, [pos], v, mask=mb)
            plsc.store_scatter(cidx_v, [pos], i * 16 + iota, mask=mb)
            return w + incl[15]

        n_cur = lax.fori_loop(0, _NV, hb, jnp.int32(0))
        plsc.store_scatter(cand_v, [n_cur + iota], zeros, mask=None)
        plsc.store_scatter(cidx_v, [n_cur + iota], padidx, mask=None)

        for rnd in range(6):
            sh = 19 - 4 * rnd if rnd < 5 else 0
            msk = jnp.int32(15) if rnd < 5 else jnp.int32(7)
            final = rnd == 5
            nv_cur = (n_cur + 15) >> 4

            for l in range(16):
                hist16[pl.ds(l * 16, 16)] = zeros

            def hr(i, carry, sh=sh, msk=msk):
                u = cand_v[pl.ds(i * 16, 16)] & jnp.int32(_ABS)
                d = (u >> sh) & msk
                plsc.addupdate_scatter(hist16, [iota16 + d], ones)
                return carry

            lax.fori_loop(0, nv_cur, hr, jnp.int32(0))
            d_r, k_rem = choose16(k_rem)

            if final:

                def fr(i, carry, sh=sh, msk=msk, d_r=d_r):
                    v = cand_v[pl.ds(i * 16, 16)]
                    idx = cidx_v[pl.ds(i * 16, 16)]
                    u = v & jnp.int32(_ABS)
                    d = (u >> sh) & msk
                    plsc.store_scatter(x_v, [idx], v, mask=d >= d_r)
                    return carry

                lax.fori_loop(0, nv_cur, fr, jnp.int32(0))
            else:

                def cr(i, w, sh=sh, msk=msk, d_r=d_r):
                    v = cand_v[pl.ds(i * 16, 16)]
                    idx = cidx_v[pl.ds(i * 16, 16)]
                    u = v & jnp.int32(_ABS)
                    d = (u >> sh) & msk
                    plsc.store_scatter(x_v, [idx], v, mask=d > d_r)
                    mi = (d == d_r).astype(jnp.int32)
                    incl = plsc.cumsum(mi)
                    pos = w + incl - mi
                    mb = mi == 1
                    plsc.store_scatter(cand_v, [pos], v, mask=mb)
                    plsc.store_scatter(cidx_v, [pos], idx, mask=mb)
                    return w + incl[15]

                n_cur = lax.fori_loop(0, nv_cur, cr, jnp.int32(0))
                plsc.store_scatter(cand_v, [n_cur + iota], zeros, mask=None)
                plsc.store_scatter(cidx_v, [n_cur + iota], padidx, mask=None)

        pltpu.sync_copy(x_v.at[pl.ds(0, _N)], o_hbm.at[row])


def _sc_call(xi):
    mesh = plsc.VectorSubcoreMesh(
        core_axis_name="c", subcore_axis_name="s", num_cores=_NC, num_subcores=_NS
    )
    return pl.kernel(
        _sc_body,
        out_type=jax.ShapeDtypeStruct((_M_SC, _N), jnp.int32),
        mesh=mesh,
        compiler_params=pltpu.CompilerParams(needs_layout_passes=False),
        scratch_types=[
            pltpu.VMEM((_N + 16,), jnp.int32),
            pltpu.VMEM((_N + 16,), jnp.int32),
            pltpu.VMEM((_N + 16,), jnp.int32),
            pltpu.VMEM((4096,), jnp.int32),
            pltpu.VMEM((256,), jnp.int32),
            pltpu.SemaphoreType.DMA,
        ],
    )(xi)


def _tc_block_body(x_ref, o_ref):
    r, n = x_ref.shape
    xb = x_ref[...]
    u = lax.bitcast_convert_type(xb, jnp.int32) & jnp.int32(_ABS)
    u3 = u.reshape(r, n // 128, 128)

    def _counts3(t3, sh):
        c1 = t3 | (jnp.int32(1) << sh)
        c2 = t3 | (jnp.int32(2) << sh)
        c3 = t3 | (jnp.int32(3) << sh)
        f = (
            (u3 >= c1).astype(jnp.int32)
            + jnp.where(u3 >= c2, jnp.int32(1 << 10), 0)
            + jnp.where(u3 >= c3, jnp.int32(1 << 20), 0)
        )
        s = jnp.sum(f, axis=1)
        cnt1 = jnp.sum(s & 1023, axis=-1).reshape(r, 1, 1)
        cnt2 = jnp.sum((s >> 10) & 1023, axis=-1).reshape(r, 1, 1)
        cnt3 = jnp.sum(s >> 20, axis=-1).reshape(r, 1, 1)
        return c1, c2, c3, cnt1, cnt2, cnt3

    t3 = jnp.zeros((r, 1, 1), jnp.int32)
    for i in range(15):
        sh = 29 - 2 * i
        c1, c2, c3, cnt1, cnt2, cnt3 = _counts3(t3, sh)
        t3 = jnp.where(
            cnt3 >= _K,
            c3,
            jnp.where(cnt2 >= _K, c2, jnp.where(cnt1 >= _K, c1, t3)),
        )
    cand = t3 | jnp.int32(1)
    cnt = jnp.sum((u3 >= cand).astype(jnp.int32), axis=(1, 2)).reshape(r, 1, 1)
    t3 = jnp.where(cnt >= _K, cand, t3)
    t = t3.reshape(r, 1)
    o_ref[...] = jnp.where(u >= t, xb, jnp.float32(0.0))


def _tc_call(x):
    m, n = x.shape
    r = _TC_BLOCK
    off = _M_SC // r
    return pl.pallas_call(
        _tc_block_body,
        grid=(m // r,),
        in_specs=[pl.BlockSpec((r, n), lambda i: (i, 0))],
        out_specs=pl.BlockSpec((r, n), lambda i: (i + off, 0)),
        out_shape=jax.ShapeDtypeStruct((_M, n), x.dtype),
    )(x)


def kernel(x):
    xi_sc = lax.bitcast_convert_type(x[:_M_SC], jnp.int32)
    o_sc = lax.bitcast_convert_type(_sc_call(xi_sc), jnp.float32)
    o_tc = _tc_call(x[_M_SC:])
    return lax.dynamic_update_slice(o_tc, o_sc, (0, 0))
